# baseline (speedup 1.0000x reference)
import functools

import jax
import jax.numpy as jnp
from jax import lax
from jax.experimental import pallas as pl
from jax.experimental.pallas import tpu as pltpu
from jax.experimental.pallas import tpu_sc as plsc

N_NODES = 10000
D = 128
E = 320000

NC = 2
NS = 16
NW = NC * NS
CH = 64
NBUF = 4
BLK_CH = 16
BLK = BLK_CH * CH
N_CHUNKS = 160
N_BLOCKS = N_CHUNKS // BLK_CH
EDGES_PER_TILE = N_CHUNKS * CH
E_PAD = NW * EDGES_PER_TILE
N_SLABS = 157
N_PAD = N_SLABS * CH
DEN_SLAB = 128
N_PAD_DEN = 10368
N_DEN_SLABS = N_PAD_DEN // DEN_SLAB

ROWS_BLK = 1000


def _tc_transform_body(x_ref, w_ref, a_ref, h_ref, s_ref):
    h = jnp.dot(x_ref[...], w_ref[...], preferred_element_type=jnp.float32)
    h_ref[...] = h
    s_ref[...] = jnp.dot(h, a_ref[...], preferred_element_type=jnp.float32)


def _tc_transform(x, w, a_mat):
    n_blocks = N_NODES // ROWS_BLK
    return pl.pallas_call(
        _tc_transform_body,
        grid=(n_blocks,),
        in_specs=[
            pl.BlockSpec((ROWS_BLK, D), lambda i: (i, 0)),
            pl.BlockSpec((D, D), lambda i: (0, 0)),
            pl.BlockSpec((D, D), lambda i: (0, 0)),
        ],
        out_specs=[
            pl.BlockSpec((ROWS_BLK, D), lambda i: (i, 0)),
            pl.BlockSpec((ROWS_BLK, D), lambda i: (i, 0)),
        ],
        out_shape=[
            jax.ShapeDtypeStruct((N_NODES, D), jnp.float32),
            jax.ShapeDtypeStruct((N_NODES, D), jnp.float32),
        ],
    )(x, w, a_mat)


def _sc_body(src_hbm, dst_hbm, h_hbm, s1_hbm, s2_hbm,
             outp_hbm, denp_hbm,
             srcb, dstb, rows, s1g, s2g, wbuf, zrow, out_sh, den_sh,
             gsem0, gsem1, gsem2, gsem3, ssem0, ssem1, ssem2, ssem3):
    gsem = (gsem0, gsem1, gsem2, gsem3)
    ssem = (ssem0, ssem1, ssem2, ssem3)
    cid = lax.axis_index("c")
    sid = lax.axis_index("s")
    wid = sid * NC + cid

    zero16 = jnp.zeros((16,), jnp.float32)

    def _zero_rows(r, _):
        for k in range(D // 16):
            rows[0, r, pl.ds(k * 16, 16)] = zero16
        return 0

    lax.fori_loop(0, CH, _zero_rows, 0)
    for j in range(DEN_SLAB // 16):
        zrow[pl.ds(j * 16, 16)] = zero16

    for k in range((N_SLABS + NS - 1) // NS):
        s = sid + NS * k

        @pl.when(s < N_SLABS)
        def _():
            pltpu.sync_copy(rows.at[0], out_sh.at[pl.ds(s * CH, CH)])

    for k in range((N_DEN_SLABS + NS - 1) // NS):
        s = sid + NS * k

        @pl.when(s < N_DEN_SLABS)
        def _():
            pltpu.sync_copy(zrow, den_sh.at[pl.ds(s * DEN_SLAB, DEN_SLAB)])

    plsc.subcore_barrier()

    ebase0 = wid * EDGES_PER_TILE

    def _stage(bb, krow, b):
        pltpu.async_copy(s1_hbm.at[srcb.at[bb, krow]], s1g.at[b], gsem[b])
        pltpu.async_copy(s2_hbm.at[dstb.at[bb, krow]], s2g.at[b], gsem[b])
        pltpu.async_copy(h_hbm.at[dstb.at[bb, krow]], rows.at[b], gsem[b])

    def _drain_gather(b):
        pltpu.make_async_copy(s1_hbm.at[pl.ds(0, CH)], s1g.at[b],
                              gsem[b]).wait()
        pltpu.make_async_copy(s2_hbm.at[pl.ds(0, CH)], s2g.at[b],
                              gsem[b]).wait()
        pltpu.make_async_copy(h_hbm.at[pl.ds(0, CH)], rows.at[b],
                              gsem[b]).wait()

    def _wait_scatter(b):
        pltpu.make_async_copy(wbuf.at[b], den_sh.at[pl.ds(0, CH)],
                              ssem[b]).wait()
        pltpu.make_async_copy(rows.at[b], out_sh.at[pl.ds(0, CH)],
                              ssem[b]).wait()

    def _process(c, bb, krow, b):
        base = ebase0 + c * CH
        _drain_gather(b)

        iota16 = lax.iota(jnp.int32, 16)
        for j in range(CH // 16):
            sl = pl.ds(j * 16, 16)
            z = s1g[b, sl] + s2g[b, sl]
            z = jnp.where(z >= 0.0, z, 0.2 * z)
            z = jnp.clip(z, -2.0, 2.0)
            w = jnp.exp(z)
            eid = base + j * 16 + iota16
            w = jnp.where(eid < E, w, 0.0)
            wbuf[b, sl] = w

        def _scale(i, _):
            for dr in range(8):
                r = i * 8 + dr
                wv = plsc.load_gather(wbuf.at[b],
                                      [jnp.full((16,), r, jnp.int32)])
                for k in range(D // 16):
                    sl = pl.ds(k * 16, 16)
                    rows[b, r, sl] = rows[b, r, sl] * wv
            return 0

        lax.fori_loop(0, CH // 8, _scale, 0)

        pltpu.async_copy(wbuf.at[b], den_sh.at[srcb.at[bb, krow]], ssem[b],
                         add=True)
        pltpu.async_copy(rows.at[b], out_sh.at[srcb.at[bb, krow]], ssem[b],
                         add=True)

    ebrow = wid * N_CHUNKS
    pltpu.sync_copy(src_hbm.at[pl.ds(ebrow, BLK_CH)], srcb.at[0])
    pltpu.sync_copy(dst_hbm.at[pl.ds(ebrow, BLK_CH)], dstb.at[0])
    _stage(0, 0, 0)

    def _block(g2, _):
        for gg in range(2):
            g = g2 * 2 + gg
            bb = gg
            nbb = 1 - gg
            for k in range(BLK_CH):
                c = g * BLK_CH + k
                b = k % NBUF
                nb = (k + 1) % NBUF
                if gg == 0 and k < 3:
                    @pl.when(g2 >= 1)
                    def _():
                        _wait_scatter(nb)
                else:
                    _wait_scatter(nb)
                if k == 2:
                    brow = ebrow + (g + 1) * BLK_CH
                    pltpu.sync_copy(src_hbm.at[pl.ds(brow, BLK_CH)],
                                    srcb.at[nbb])
                    pltpu.sync_copy(dst_hbm.at[pl.ds(brow, BLK_CH)],
                                    dstb.at[nbb])
                if k < BLK_CH - 1:
                    _stage(bb, k + 1, nb)
                else:
                    _stage(nbb, 0, nb)
                _process(c, bb, k, b)
        return 0

    lax.fori_loop(0, N_BLOCKS // 2, _block, 0)

    _wait_scatter((N_CHUNKS - 3) % NBUF)
    _wait_scatter((N_CHUNKS - 2) % NBUF)
    _wait_scatter((N_CHUNKS - 1) % NBUF)
    _drain_gather(N_CHUNKS % NBUF)

    plsc.subcore_barrier()

    for k in range((N_SLABS + NS - 1) // NS):
        s = sid + NS * k

        @pl.when(s < N_SLABS)
        def _():
            pltpu.sync_copy(out_sh.at[pl.ds(s * CH, CH)],
                            outp_hbm.at[cid, pl.ds(s * CH, CH)])

    for k in range((N_DEN_SLABS + NS - 1) // NS):
        s = sid + NS * k

        @pl.when(s < N_DEN_SLABS)
        def _():
            pltpu.sync_copy(den_sh.at[pl.ds(s * DEN_SLAB, DEN_SLAB)],
                            denp_hbm.at[cid, pl.ds(s * DEN_SLAB, DEN_SLAB)])


def _sc_aggregate(src, dst, h, s1, s2):
    mesh = plsc.VectorSubcoreMesh(core_axis_name="c", subcore_axis_name="s")
    fn = pl.kernel(
        _sc_body,
        out_type=[
            jax.ShapeDtypeStruct((NC, N_PAD, D), jnp.float32),
            jax.ShapeDtypeStruct((NC, N_PAD_DEN), jnp.float32),
        ],
        mesh=mesh,
        scratch_types=[
            pltpu.VMEM((2, BLK_CH, CH), jnp.int32),
            pltpu.VMEM((2, BLK_CH, CH), jnp.int32),
            pltpu.VMEM((NBUF, CH, D), jnp.float32),
            pltpu.VMEM((NBUF, CH), jnp.float32),
            pltpu.VMEM((NBUF, CH), jnp.float32),
            pltpu.VMEM((NBUF, CH), jnp.float32),
            pltpu.VMEM((DEN_SLAB,), jnp.float32),
            pltpu.VMEM_SHARED((N_PAD, D), jnp.float32),
            pltpu.VMEM_SHARED((N_PAD_DEN,), jnp.float32),
            pltpu.SemaphoreType.DMA,
            pltpu.SemaphoreType.DMA,
            pltpu.SemaphoreType.DMA,
            pltpu.SemaphoreType.DMA,
            pltpu.SemaphoreType.DMA,
            pltpu.SemaphoreType.DMA,
            pltpu.SemaphoreType.DMA,
            pltpu.SemaphoreType.DMA,
        ],
        compiler_params=pltpu.CompilerParams(needs_layout_passes=False),
    )
    return fn(src, dst, h, s1, s2)


def _tc_combine_body(o0_ref, o1_ref, d0_ref, d1_ref, out_ref):
    den = d0_ref[...] + d1_ref[...]
    num = o0_ref[...] + o1_ref[...]
    out_ref[...] = jnp.where(den > 0.0, num / jnp.where(den > 0.0, den, 1.0),
                             0.0)


def _tc_combine(o0, o1, d0, d1):
    n_blocks = N_NODES // ROWS_BLK
    return pl.pallas_call(
        _tc_combine_body,
        grid=(n_blocks,),
        in_specs=[
            pl.BlockSpec((ROWS_BLK, D), lambda i: (i, 0)),
            pl.BlockSpec((ROWS_BLK, D), lambda i: (i, 0)),
            pl.BlockSpec((ROWS_BLK, 1), lambda i: (i, 0)),
            pl.BlockSpec((ROWS_BLK, 1), lambda i: (i, 0)),
        ],
        out_specs=pl.BlockSpec((ROWS_BLK, D), lambda i: (i, 0)),
        out_shape=jax.ShapeDtypeStruct((N_NODES, D), jnp.float32),
    )(o0, o1, d0, d1)


def kernel(node_states, edges, kernel, kernel_attention):
    w = kernel.astype(jnp.float32)
    a = kernel_attention.astype(jnp.float32)
    a_mat = jnp.pad(jnp.concatenate([a[:D], a[D:]], axis=1),
                    ((0, 0), (0, D - 2)))

    edges_i = edges.astype(jnp.int32)
    pad = E_PAD - E + BLK + CH
    src = jnp.concatenate([edges_i[:, 0],
                           jnp.zeros((pad,), jnp.int32)]).reshape(-1, CH)
    dst = jnp.concatenate([edges_i[:, 1],
                           jnp.zeros((pad,), jnp.int32)]).reshape(-1, CH)

    h, s_cat = _tc_transform(node_states.astype(jnp.float32), w, a_mat)
    s1 = s_cat[:, 0]
    s2 = s_cat[:, 1]

    outp, denp = _sc_aggregate(src, dst, h, s1, s2)

    return _tc_combine(outp[0, :N_NODES], outp[1, :N_NODES],
                       denp[0, :N_NODES, None], denp[1, :N_NODES, None])


# speedup vs baseline: 2.2075x; 2.2075x over previous
import functools

import jax
import jax.numpy as jnp
from jax import lax
from jax.experimental import pallas as pl
from jax.experimental.pallas import tpu as pltpu
from jax.experimental.pallas import tpu_sc as plsc

N_NODES = 10000
D = 128
E = 320000

NC = 2
NS = 16
NW = NC * NS
CH = 64
NBUF = 4
BLK_CH = 16
BLK = BLK_CH * CH
N_CHUNKS = 160
N_BLOCKS = N_CHUNKS // BLK_CH
EDGES_PER_TILE = N_CHUNKS * CH
E_PAD = NW * EDGES_PER_TILE
E_REAL_PT = E // NW
N_SLABS = 157
N_PAD = N_SLABS * CH
DEN_SLAB = 128
N_PAD_DEN = 10368
N_DEN_SLABS = N_PAD_DEN // DEN_SLAB

ROWS_BLK = 1000


def _tc_transform_body(x_ref, w_ref, a_ref, h_ref, s_ref):
    h = jnp.dot(x_ref[...], w_ref[...], preferred_element_type=jnp.float32)
    h_ref[...] = h
    s_ref[...] = jnp.dot(h, a_ref[...], preferred_element_type=jnp.float32)


def _tc_transform(x, w, a_mat):
    n_blocks = N_NODES // ROWS_BLK
    return pl.pallas_call(
        _tc_transform_body,
        grid=(n_blocks,),
        in_specs=[
            pl.BlockSpec((ROWS_BLK, D), lambda i: (i, 0)),
            pl.BlockSpec((D, D), lambda i: (0, 0)),
            pl.BlockSpec((D, D), lambda i: (0, 0)),
        ],
        out_specs=[
            pl.BlockSpec((ROWS_BLK, D), lambda i: (i, 0)),
            pl.BlockSpec((ROWS_BLK, D), lambda i: (i, 0)),
        ],
        out_shape=[
            jax.ShapeDtypeStruct((N_NODES, D), jnp.float32),
            jax.ShapeDtypeStruct((N_NODES, D), jnp.float32),
        ],
    )(x, w, a_mat)


def _sc_body(src_hbm, dst_hbm, h_hbm, s1_hbm, s2_hbm,
             outp_hbm, denp_hbm,
             srcb, dstb, rows, s1g, s2g, wbuf, zrow, out_sh, den_sh,
             gsem0, gsem1, gsem2, gsem3, ssem0, ssem1, ssem2, ssem3):
    gsem = (gsem0, gsem1, gsem2, gsem3)
    ssem = (ssem0, ssem1, ssem2, ssem3)
    cid = lax.axis_index("c")
    sid = lax.axis_index("s")
    wid = sid * NC + cid

    zero16 = jnp.zeros((16,), jnp.float32)

    def _zero_rows(r, _):
        for k in range(D // 16):
            rows[0, r, pl.ds(k * 16, 16)] = zero16
        return 0

    lax.fori_loop(0, CH, _zero_rows, 0)
    for j in range(DEN_SLAB // 16):
        zrow[pl.ds(j * 16, 16)] = zero16

    for k in range((N_SLABS + NS - 1) // NS):
        s = sid + NS * k

        @pl.when(s < N_SLABS)
        def _():
            pltpu.sync_copy(rows.at[0], out_sh.at[pl.ds(s * CH, CH)])

    for k in range((N_DEN_SLABS + NS - 1) // NS):
        s = sid + NS * k

        @pl.when(s < N_DEN_SLABS)
        def _():
            pltpu.sync_copy(zrow, den_sh.at[pl.ds(s * DEN_SLAB, DEN_SLAB)])

    plsc.subcore_barrier()


    def _stage(bb, krow, b):
        pltpu.async_copy(s1_hbm.at[srcb.at[bb, krow]], s1g.at[b], gsem[b])
        pltpu.async_copy(s2_hbm.at[dstb.at[bb, krow]], s2g.at[b], gsem[b])
        pltpu.async_copy(h_hbm.at[dstb.at[bb, krow]], rows.at[b], gsem[b])

    def _drain_gather(b):
        pltpu.make_async_copy(s1_hbm.at[pl.ds(0, CH)], s1g.at[b],
                              gsem[b]).wait()
        pltpu.make_async_copy(s2_hbm.at[pl.ds(0, CH)], s2g.at[b],
                              gsem[b]).wait()
        pltpu.make_async_copy(h_hbm.at[pl.ds(0, CH)], rows.at[b],
                              gsem[b]).wait()

    def _wait_scatter(b):
        pltpu.make_async_copy(wbuf.at[b], den_sh.at[pl.ds(0, CH)],
                              ssem[b]).wait()
        pltpu.make_async_copy(rows.at[b], out_sh.at[pl.ds(0, CH)],
                              ssem[b]).wait()

    def _process(c, bb, krow, b):
        base = c * CH
        _drain_gather(b)

        iota16 = lax.iota(jnp.int32, 16)
        for j in range(CH // 16):
            sl = pl.ds(j * 16, 16)
            z = s1g[b, sl] + s2g[b, sl]
            z = jnp.where(z >= 0.0, z, 0.2 * z)
            z = jnp.clip(z, -2.0, 2.0)
            w = jnp.exp(z)
            eid = base + j * 16 + iota16
            w = jnp.where(eid < E_REAL_PT, w, 0.0)
            wbuf[b, sl] = w

        def _scale(i, _):
            for dr in range(4):
                r = i * 4 + dr
                wv = plsc.load_gather(wbuf.at[b],
                                      [jnp.full((16,), r, jnp.int32)])
                for k in range(D // 16):
                    sl = pl.ds(k * 16, 16)
                    rows[b, r, sl] = rows[b, r, sl] * wv
            return 0

        lax.fori_loop(0, CH // 4, _scale, 0)

        pltpu.async_copy(wbuf.at[b], den_sh.at[srcb.at[bb, krow]], ssem[b],
                         add=True)
        pltpu.async_copy(rows.at[b], out_sh.at[srcb.at[bb, krow]], ssem[b],
                         add=True)

    ebrow = wid * N_CHUNKS
    pltpu.sync_copy(src_hbm.at[pl.ds(ebrow, BLK_CH)], srcb.at[0])
    pltpu.sync_copy(dst_hbm.at[pl.ds(ebrow, BLK_CH)], dstb.at[0])
    _stage(0, 0, 0)

    def _block(g2, _):
        for gg in range(2):
            g = g2 * 2 + gg
            bb = gg
            nbb = 1 - gg
            for k in range(BLK_CH):
                c = g * BLK_CH + k
                b = k % NBUF
                nb = (k + 1) % NBUF
                if gg == 0 and k < 3:
                    @pl.when(g2 >= 1)
                    def _():
                        _wait_scatter(nb)
                else:
                    _wait_scatter(nb)
                if k == 2:
                    brow = ebrow + (g + 1) * BLK_CH
                    pltpu.sync_copy(src_hbm.at[pl.ds(brow, BLK_CH)],
                                    srcb.at[nbb])
                    pltpu.sync_copy(dst_hbm.at[pl.ds(brow, BLK_CH)],
                                    dstb.at[nbb])
                if k < BLK_CH - 1:
                    _stage(bb, k + 1, nb)
                else:
                    _stage(nbb, 0, nb)
                _process(c, bb, k, b)
        return 0

    lax.fori_loop(0, N_BLOCKS // 2, _block, 0)

    _wait_scatter((N_CHUNKS - 3) % NBUF)
    _wait_scatter((N_CHUNKS - 2) % NBUF)
    _wait_scatter((N_CHUNKS - 1) % NBUF)
    _drain_gather(N_CHUNKS % NBUF)

    plsc.subcore_barrier()

    for k in range((N_SLABS + NS - 1) // NS):
        s = sid + NS * k

        @pl.when(s < N_SLABS)
        def _():
            pltpu.sync_copy(out_sh.at[pl.ds(s * CH, CH)],
                            outp_hbm.at[cid, pl.ds(s * CH, CH)])

    for k in range((N_DEN_SLABS + NS - 1) // NS):
        s = sid + NS * k

        @pl.when(s < N_DEN_SLABS)
        def _():
            pltpu.sync_copy(den_sh.at[pl.ds(s * DEN_SLAB, DEN_SLAB)],
                            denp_hbm.at[cid, pl.ds(s * DEN_SLAB, DEN_SLAB)])


def _sc_aggregate(src, dst, h, s1, s2):
    mesh = plsc.VectorSubcoreMesh(core_axis_name="c", subcore_axis_name="s")
    fn = pl.kernel(
        _sc_body,
        out_type=[
            jax.ShapeDtypeStruct((NC, N_PAD, D), jnp.float32),
            jax.ShapeDtypeStruct((NC, N_PAD_DEN), jnp.float32),
        ],
        mesh=mesh,
        scratch_types=[
            pltpu.VMEM((2, BLK_CH, CH), jnp.int32),
            pltpu.VMEM((2, BLK_CH, CH), jnp.int32),
            pltpu.VMEM((NBUF, CH, D), jnp.float32),
            pltpu.VMEM((NBUF, CH), jnp.float32),
            pltpu.VMEM((NBUF, CH), jnp.float32),
            pltpu.VMEM((NBUF, CH), jnp.float32),
            pltpu.VMEM((DEN_SLAB,), jnp.float32),
            pltpu.VMEM_SHARED((N_PAD, D), jnp.float32),
            pltpu.VMEM_SHARED((N_PAD_DEN,), jnp.float32),
            pltpu.SemaphoreType.DMA,
            pltpu.SemaphoreType.DMA,
            pltpu.SemaphoreType.DMA,
            pltpu.SemaphoreType.DMA,
            pltpu.SemaphoreType.DMA,
            pltpu.SemaphoreType.DMA,
            pltpu.SemaphoreType.DMA,
            pltpu.SemaphoreType.DMA,
        ],
        compiler_params=pltpu.CompilerParams(needs_layout_passes=False),
    )
    return fn(src, dst, h, s1, s2)


def _tc_combine_body(o0_ref, o1_ref, d0_ref, d1_ref, out_ref):
    den = d0_ref[...] + d1_ref[...]
    num = o0_ref[...] + o1_ref[...]
    out_ref[...] = jnp.where(den > 0.0, num / jnp.where(den > 0.0, den, 1.0),
                             0.0)


def _tc_combine(o0, o1, d0, d1):
    n_blocks = N_NODES // ROWS_BLK
    return pl.pallas_call(
        _tc_combine_body,
        grid=(n_blocks,),
        in_specs=[
            pl.BlockSpec((ROWS_BLK, D), lambda i: (i, 0)),
            pl.BlockSpec((ROWS_BLK, D), lambda i: (i, 0)),
            pl.BlockSpec((ROWS_BLK, 1), lambda i: (i, 0)),
            pl.BlockSpec((ROWS_BLK, 1), lambda i: (i, 0)),
        ],
        out_specs=pl.BlockSpec((ROWS_BLK, D), lambda i: (i, 0)),
        out_shape=jax.ShapeDtypeStruct((N_NODES, D), jnp.float32),
    )(o0, o1, d0, d1)


def kernel(node_states, edges, kernel, kernel_attention):
    w = kernel.astype(jnp.float32)
    a = kernel_attention.astype(jnp.float32)
    a_mat = jnp.pad(jnp.concatenate([a[:D], a[D:]], axis=1),
                    ((0, 0), (0, D - 2)))

    edges_i = edges.astype(jnp.int32)
    pad_pt = EDGES_PER_TILE - E_REAL_PT
    dummy = (jnp.arange(NW * pad_pt, dtype=jnp.int32) % N_NODES)
    dummy = dummy.reshape(NW, pad_pt)

    def _tile_layout(col):
        per_tile = jnp.concatenate([col.reshape(NW, E_REAL_PT), dummy], 1)
        tail = jnp.zeros(((BLK_CH + 1) * CH,), jnp.int32)
        return jnp.concatenate([per_tile.reshape(-1), tail]).reshape(-1, CH)

    src = _tile_layout(edges_i[:, 0])
    dst = _tile_layout(edges_i[:, 1])

    h, s_cat = _tc_transform(node_states.astype(jnp.float32), w, a_mat)
    s1 = s_cat[:, 0]
    s2 = s_cat[:, 1]

    outp, denp = _sc_aggregate(src, dst, h, s1, s2)

    return _tc_combine(outp[0, :N_NODES], outp[1, :N_NODES],
                       denp[0, :N_NODES, None], denp[1, :N_NODES, None])


# speedup vs baseline: 2.2455x; 1.0172x over previous
import functools

import jax
import jax.numpy as jnp
from jax import lax
from jax.experimental import pallas as pl
from jax.experimental.pallas import tpu as pltpu
from jax.experimental.pallas import tpu_sc as plsc

N_NODES = 10000
D = 128
E = 320000

NC = 2
NS = 16
NW = NC * NS
CH = 64
NBUF = 4
BLK_CH = 16
BLK = BLK_CH * CH
N_CHUNKS = 160
N_BLOCKS = N_CHUNKS // BLK_CH
EDGES_PER_TILE = N_CHUNKS * CH
E_PAD = NW * EDGES_PER_TILE
E_REAL_PT = E // NW
N_SLABS = 157
N_PAD = N_SLABS * CH
DEN_SLAB = 128
N_PAD_DEN = 10368
N_DEN_SLABS = N_PAD_DEN // DEN_SLAB

ROWS_BLK = 1000


def _tc_transform_body(x_ref, w_ref, a_ref, h_ref, s_ref):
    h = jnp.dot(x_ref[...], w_ref[...], preferred_element_type=jnp.float32)
    h_ref[...] = h
    s_ref[...] = jnp.dot(h, a_ref[...], preferred_element_type=jnp.float32)


def _tc_transform(x, w, a_mat):
    n_blocks = N_NODES // ROWS_BLK
    return pl.pallas_call(
        _tc_transform_body,
        grid=(n_blocks,),
        in_specs=[
            pl.BlockSpec((ROWS_BLK, D), lambda i: (i, 0)),
            pl.BlockSpec((D, D), lambda i: (0, 0)),
            pl.BlockSpec((D, D), lambda i: (0, 0)),
        ],
        out_specs=[
            pl.BlockSpec((ROWS_BLK, D), lambda i: (i, 0)),
            pl.BlockSpec((ROWS_BLK, D), lambda i: (i, 0)),
        ],
        out_shape=[
            jax.ShapeDtypeStruct((N_NODES, D), jnp.float32),
            jax.ShapeDtypeStruct((N_NODES, D), jnp.float32),
        ],
    )(x, w, a_mat)


def _sc_body(src_hbm, dst_hbm, h_hbm, s1_hbm, s2_hbm,
             outp_hbm, denp_hbm,
             srcb, dstb, rows, s1g, s2g, wbuf, zrow, out_sh, den_sh,
             gsem0, gsem1, gsem2, gsem3, ssem0, ssem1, ssem2, ssem3):
    gsem = (gsem0, gsem1, gsem2, gsem3)
    ssem = (ssem0, ssem1, ssem2, ssem3)
    cid = lax.axis_index("c")
    sid = lax.axis_index("s")
    wid = sid * NC + cid

    zero16 = jnp.zeros((16,), jnp.float32)

    def _zero_rows(r, _):
        for k in range(D // 16):
            rows[0, r, pl.ds(k * 16, 16)] = zero16
        return 0

    lax.fori_loop(0, CH, _zero_rows, 0)
    for j in range(DEN_SLAB // 16):
        zrow[pl.ds(j * 16, 16)] = zero16

    for k in range((N_SLABS + NS - 1) // NS):
        s = sid + NS * k

        @pl.when(s < N_SLABS)
        def _():
            pltpu.async_copy(rows.at[0], out_sh.at[pl.ds(s * CH, CH)],
                             gsem0)

    for k in range((N_DEN_SLABS + NS - 1) // NS):
        s = sid + NS * k

        @pl.when(s < N_DEN_SLABS)
        def _():
            pltpu.async_copy(zrow, den_sh.at[pl.ds(s * DEN_SLAB, DEN_SLAB)],
                             gsem1)

    for k in range((N_SLABS + NS - 1) // NS):
        s = sid + NS * k

        @pl.when(s < N_SLABS)
        def _():
            pltpu.make_async_copy(rows.at[0], out_sh.at[pl.ds(s * CH, CH)],
                                  gsem0).wait()

    for k in range((N_DEN_SLABS + NS - 1) // NS):
        s = sid + NS * k

        @pl.when(s < N_DEN_SLABS)
        def _():
            pltpu.make_async_copy(zrow,
                                  den_sh.at[pl.ds(s * DEN_SLAB, DEN_SLAB)],
                                  gsem1).wait()

    plsc.subcore_barrier()


    def _stage(bb, krow, b):
        pltpu.async_copy(s1_hbm.at[srcb.at[bb, krow]], s1g.at[b], gsem[b])
        pltpu.async_copy(s2_hbm.at[dstb.at[bb, krow]], s2g.at[b], gsem[b])
        pltpu.async_copy(h_hbm.at[dstb.at[bb, krow]], rows.at[b], gsem[b])

    def _drain_gather(b):
        pltpu.make_async_copy(s1_hbm.at[pl.ds(0, CH)], s1g.at[b],
                              gsem[b]).wait()
        pltpu.make_async_copy(s2_hbm.at[pl.ds(0, CH)], s2g.at[b],
                              gsem[b]).wait()
        pltpu.make_async_copy(h_hbm.at[pl.ds(0, CH)], rows.at[b],
                              gsem[b]).wait()

    def _wait_scatter(b):
        pltpu.make_async_copy(wbuf.at[b], den_sh.at[pl.ds(0, CH)],
                              ssem[b]).wait()
        pltpu.make_async_copy(rows.at[b], out_sh.at[pl.ds(0, CH)],
                              ssem[b]).wait()

    def _process(c, bb, krow, b):
        base = c * CH
        _drain_gather(b)

        iota16 = lax.iota(jnp.int32, 16)
        for j in range(CH // 16):
            sl = pl.ds(j * 16, 16)
            z = s1g[b, sl] + s2g[b, sl]
            z = jnp.where(z >= 0.0, z, 0.2 * z)
            z = jnp.clip(z, -2.0, 2.0)
            w = jnp.exp(z)
            eid = base + j * 16 + iota16
            w = jnp.where(eid < E_REAL_PT, w, 0.0)
            wbuf[b, sl] = w

        def _scale(i, _):
            for dr in range(4):
                r = i * 4 + dr
                wv = plsc.load_gather(wbuf.at[b],
                                      [jnp.full((16,), r, jnp.int32)])
                for k in range(D // 16):
                    sl = pl.ds(k * 16, 16)
                    rows[b, r, sl] = rows[b, r, sl] * wv
            return 0

        lax.fori_loop(0, CH // 4, _scale, 0)

        pltpu.async_copy(wbuf.at[b], den_sh.at[srcb.at[bb, krow]], ssem[b],
                         add=True)
        pltpu.async_copy(rows.at[b], out_sh.at[srcb.at[bb, krow]], ssem[b],
                         add=True)

    ebrow = wid * N_CHUNKS
    pltpu.sync_copy(src_hbm.at[pl.ds(ebrow, BLK_CH)], srcb.at[0])
    pltpu.sync_copy(dst_hbm.at[pl.ds(ebrow, BLK_CH)], dstb.at[0])
    _stage(0, 0, 0)

    def _block(g2, _):
        for gg in range(2):
            g = g2 * 2 + gg
            bb = gg
            nbb = 1 - gg
            for k in range(BLK_CH):
                c = g * BLK_CH + k
                b = k % NBUF
                nb = (k + 1) % NBUF
                if gg == 0 and k < 3:
                    @pl.when(g2 >= 1)
                    def _():
                        _wait_scatter(nb)
                else:
                    _wait_scatter(nb)
                if k == 2:
                    brow = ebrow + (g + 1) * BLK_CH
                    pltpu.sync_copy(src_hbm.at[pl.ds(brow, BLK_CH)],
                                    srcb.at[nbb])
                    pltpu.sync_copy(dst_hbm.at[pl.ds(brow, BLK_CH)],
                                    dstb.at[nbb])
                if k < BLK_CH - 1:
                    _stage(bb, k + 1, nb)
                else:
                    _stage(nbb, 0, nb)
                _process(c, bb, k, b)
        return 0

    lax.fori_loop(0, N_BLOCKS // 2, _block, 0)

    _wait_scatter((N_CHUNKS - 3) % NBUF)
    _wait_scatter((N_CHUNKS - 2) % NBUF)
    _wait_scatter((N_CHUNKS - 1) % NBUF)
    _drain_gather(N_CHUNKS % NBUF)

    plsc.subcore_barrier()

    for k in range((N_SLABS + NS - 1) // NS):
        s = sid + NS * k

        @pl.when(s < N_SLABS)
        def _():
            pltpu.async_copy(out_sh.at[pl.ds(s * CH, CH)],
                             outp_hbm.at[cid, pl.ds(s * CH, CH)], gsem0)

    for k in range((N_DEN_SLABS + NS - 1) // NS):
        s = sid + NS * k

        @pl.when(s < N_DEN_SLABS)
        def _():
            pltpu.async_copy(den_sh.at[pl.ds(s * DEN_SLAB, DEN_SLAB)],
                             denp_hbm.at[cid, pl.ds(s * DEN_SLAB, DEN_SLAB)],
                             gsem1)

    for k in range((N_SLABS + NS - 1) // NS):
        s = sid + NS * k

        @pl.when(s < N_SLABS)
        def _():
            pltpu.make_async_copy(out_sh.at[pl.ds(s * CH, CH)],
                                  outp_hbm.at[cid, pl.ds(s * CH, CH)],
                                  gsem0).wait()

    for k in range((N_DEN_SLABS + NS - 1) // NS):
        s = sid + NS * k

        @pl.when(s < N_DEN_SLABS)
        def _():
            pltpu.make_async_copy(
                den_sh.at[pl.ds(s * DEN_SLAB, DEN_SLAB)],
                denp_hbm.at[cid, pl.ds(s * DEN_SLAB, DEN_SLAB)],
                gsem1).wait()


def _sc_aggregate(src, dst, h, s1, s2):
    mesh = plsc.VectorSubcoreMesh(core_axis_name="c", subcore_axis_name="s")
    fn = pl.kernel(
        _sc_body,
        out_type=[
            jax.ShapeDtypeStruct((NC, N_PAD, D), jnp.float32),
            jax.ShapeDtypeStruct((NC, N_PAD_DEN), jnp.float32),
        ],
        mesh=mesh,
        scratch_types=[
            pltpu.VMEM((2, BLK_CH, CH), jnp.int32),
            pltpu.VMEM((2, BLK_CH, CH), jnp.int32),
            pltpu.VMEM((NBUF, CH, D), jnp.float32),
            pltpu.VMEM((NBUF, CH), jnp.float32),
            pltpu.VMEM((NBUF, CH), jnp.float32),
            pltpu.VMEM((NBUF, CH), jnp.float32),
            pltpu.VMEM((DEN_SLAB,), jnp.float32),
            pltpu.VMEM_SHARED((N_PAD, D), jnp.float32),
            pltpu.VMEM_SHARED((N_PAD_DEN,), jnp.float32),
            pltpu.SemaphoreType.DMA,
            pltpu.SemaphoreType.DMA,
            pltpu.SemaphoreType.DMA,
            pltpu.SemaphoreType.DMA,
            pltpu.SemaphoreType.DMA,
            pltpu.SemaphoreType.DMA,
            pltpu.SemaphoreType.DMA,
            pltpu.SemaphoreType.DMA,
        ],
        compiler_params=pltpu.CompilerParams(needs_layout_passes=False),
    )
    return fn(src, dst, h, s1, s2)


def _tc_combine_body(o0_ref, o1_ref, d0_ref, d1_ref, out_ref):
    den = d0_ref[...] + d1_ref[...]
    num = o0_ref[...] + o1_ref[...]
    out_ref[...] = jnp.where(den > 0.0, num / jnp.where(den > 0.0, den, 1.0),
                             0.0)


def _tc_combine(o0, o1, d0, d1):
    n_blocks = N_NODES // ROWS_BLK
    return pl.pallas_call(
        _tc_combine_body,
        grid=(n_blocks,),
        in_specs=[
            pl.BlockSpec((ROWS_BLK, D), lambda i: (i, 0)),
            pl.BlockSpec((ROWS_BLK, D), lambda i: (i, 0)),
            pl.BlockSpec((ROWS_BLK, 1), lambda i: (i, 0)),
            pl.BlockSpec((ROWS_BLK, 1), lambda i: (i, 0)),
        ],
        out_specs=pl.BlockSpec((ROWS_BLK, D), lambda i: (i, 0)),
        out_shape=jax.ShapeDtypeStruct((N_NODES, D), jnp.float32),
    )(o0, o1, d0, d1)


def kernel(node_states, edges, kernel, kernel_attention):
    w = kernel.astype(jnp.float32)
    a = kernel_attention.astype(jnp.float32)
    a_mat = jnp.pad(jnp.concatenate([a[:D], a[D:]], axis=1),
                    ((0, 0), (0, D - 2)))

    edges_i = edges.astype(jnp.int32)
    pad_pt = EDGES_PER_TILE - E_REAL_PT
    dummy = (jnp.arange(NW * pad_pt, dtype=jnp.int32) % N_NODES)
    dummy = dummy.reshape(NW, pad_pt)

    def _tile_layout(col):
        per_tile = jnp.concatenate([col.reshape(NW, E_REAL_PT), dummy], 1)
        tail = jnp.zeros(((BLK_CH + 1) * CH,), jnp.int32)
        return jnp.concatenate([per_tile.reshape(-1), tail]).reshape(-1, CH)

    src = _tile_layout(edges_i[:, 0])
    dst = _tile_layout(edges_i[:, 1])

    h, s_cat = _tc_transform(node_states.astype(jnp.float32), w, a_mat)
    s1 = s_cat[:, 0]
    s2 = s_cat[:, 1]

    outp, denp = _sc_aggregate(src, dst, h, s1, s2)

    return _tc_combine(outp[0, :N_NODES], outp[1, :N_NODES],
                       denp[0, :N_NODES, None], denp[1, :N_NODES, None])


# speedup vs baseline: 2.3026x; 1.0254x over previous
import functools

import jax
import jax.numpy as jnp
from jax import lax
from jax.experimental import pallas as pl
from jax.experimental.pallas import tpu as pltpu
from jax.experimental.pallas import tpu_sc as plsc

N_NODES = 10000
D = 128
E = 320000

NC = 2
NS = 16
NW = NC * NS
CH = 80
NBUF = 4
BLK_CH = 8
BLK = BLK_CH * CH
N_CHUNKS = 128
N_BLOCKS = N_CHUNKS // BLK_CH
EDGES_PER_TILE = N_CHUNKS * CH
E_PAD = NW * EDGES_PER_TILE
E_REAL_PT = E // NW
N_SLABS = 125
N_PAD = N_SLABS * CH
DEN_SLAB = 128
N_PAD_DEN = 10368
N_DEN_SLABS = N_PAD_DEN // DEN_SLAB

ROWS_BLK = 1000


def _tc_transform_body(x_ref, w_ref, a_ref, h_ref, s_ref):
    h = jnp.dot(x_ref[...], w_ref[...], preferred_element_type=jnp.float32)
    h_ref[...] = h
    s_ref[...] = jnp.dot(h, a_ref[...], preferred_element_type=jnp.float32)


def _tc_transform(x, w, a_mat):
    n_blocks = N_NODES // ROWS_BLK
    return pl.pallas_call(
        _tc_transform_body,
        grid=(n_blocks,),
        in_specs=[
            pl.BlockSpec((ROWS_BLK, D), lambda i: (i, 0)),
            pl.BlockSpec((D, D), lambda i: (0, 0)),
            pl.BlockSpec((D, D), lambda i: (0, 0)),
        ],
        out_specs=[
            pl.BlockSpec((ROWS_BLK, D), lambda i: (i, 0)),
            pl.BlockSpec((ROWS_BLK, D), lambda i: (i, 0)),
        ],
        out_shape=[
            jax.ShapeDtypeStruct((N_NODES, D), jnp.float32),
            jax.ShapeDtypeStruct((N_NODES, D), jnp.float32),
        ],
    )(x, w, a_mat)


def _sc_body(src_hbm, dst_hbm, h_hbm, s1_hbm, s2_hbm,
             outp_hbm, denp_hbm,
             srcb, dstb, rows, s1g, s2g, wbuf, zrow, out_sh, den_sh,
             gsem0, gsem1, gsem2, gsem3, ssem0, ssem1, ssem2, ssem3):
    gsem = (gsem0, gsem1, gsem2, gsem3)
    ssem = (ssem0, ssem1, ssem2, ssem3)
    cid = lax.axis_index("c")
    sid = lax.axis_index("s")
    wid = sid * NC + cid

    zero16 = jnp.zeros((16,), jnp.float32)

    def _zero_rows(r, _):
        for k in range(D // 16):
            rows[0, r, pl.ds(k * 16, 16)] = zero16
        return 0

    lax.fori_loop(0, CH, _zero_rows, 0)
    for j in range(DEN_SLAB // 16):
        zrow[pl.ds(j * 16, 16)] = zero16

    for k in range((N_SLABS + NS - 1) // NS):
        s = sid + NS * k

        @pl.when(s < N_SLABS)
        def _():
            pltpu.async_copy(rows.at[0], out_sh.at[pl.ds(s * CH, CH)],
                             gsem0)

    for k in range((N_DEN_SLABS + NS - 1) // NS):
        s = sid + NS * k

        @pl.when(s < N_DEN_SLABS)
        def _():
            pltpu.async_copy(zrow, den_sh.at[pl.ds(s * DEN_SLAB, DEN_SLAB)],
                             gsem1)

    for k in range((N_SLABS + NS - 1) // NS):
        s = sid + NS * k

        @pl.when(s < N_SLABS)
        def _():
            pltpu.make_async_copy(rows.at[0], out_sh.at[pl.ds(s * CH, CH)],
                                  gsem0).wait()

    for k in range((N_DEN_SLABS + NS - 1) // NS):
        s = sid + NS * k

        @pl.when(s < N_DEN_SLABS)
        def _():
            pltpu.make_async_copy(zrow,
                                  den_sh.at[pl.ds(s * DEN_SLAB, DEN_SLAB)],
                                  gsem1).wait()

    plsc.subcore_barrier()


    def _stage(bb, krow, b):
        pltpu.async_copy(s1_hbm.at[srcb.at[bb, krow]], s1g.at[b], gsem[b])
        pltpu.async_copy(s2_hbm.at[dstb.at[bb, krow]], s2g.at[b], gsem[b])
        pltpu.async_copy(h_hbm.at[dstb.at[bb, krow]], rows.at[b], gsem[b])

    def _drain_gather(b):
        pltpu.make_async_copy(s1_hbm.at[pl.ds(0, CH)], s1g.at[b],
                              gsem[b]).wait()
        pltpu.make_async_copy(s2_hbm.at[pl.ds(0, CH)], s2g.at[b],
                              gsem[b]).wait()
        pltpu.make_async_copy(h_hbm.at[pl.ds(0, CH)], rows.at[b],
                              gsem[b]).wait()

    def _wait_scatter(b):
        pltpu.make_async_copy(wbuf.at[b], den_sh.at[pl.ds(0, CH)],
                              ssem[b]).wait()
        pltpu.make_async_copy(rows.at[b], out_sh.at[pl.ds(0, CH)],
                              ssem[b]).wait()

    def _process(c, bb, krow, b):
        base = c * CH
        _drain_gather(b)

        iota16 = lax.iota(jnp.int32, 16)
        for j in range(CH // 16):
            sl = pl.ds(j * 16, 16)
            z = s1g[b, sl] + s2g[b, sl]
            z = jnp.where(z >= 0.0, z, 0.2 * z)
            z = jnp.clip(z, -2.0, 2.0)
            w = jnp.exp(z)
            eid = base + j * 16 + iota16
            w = jnp.where(eid < E_REAL_PT, w, 0.0)
            wbuf[b, sl] = w

        def _scale(i, _):
            for dr in range(4):
                r = i * 4 + dr
                wv = plsc.load_gather(wbuf.at[b],
                                      [jnp.full((16,), r, jnp.int32)])
                for k in range(D // 16):
                    sl = pl.ds(k * 16, 16)
                    rows[b, r, sl] = rows[b, r, sl] * wv
            return 0

        lax.fori_loop(0, CH // 4, _scale, 0)

        pltpu.async_copy(wbuf.at[b], den_sh.at[srcb.at[bb, krow]], ssem[b],
                         add=True)
        pltpu.async_copy(rows.at[b], out_sh.at[srcb.at[bb, krow]], ssem[b],
                         add=True)

    ebrow = wid * N_CHUNKS
    pltpu.sync_copy(src_hbm.at[pl.ds(ebrow, BLK_CH)], srcb.at[0])
    pltpu.sync_copy(dst_hbm.at[pl.ds(ebrow, BLK_CH)], dstb.at[0])
    _stage(0, 0, 0)

    def _block(g2, _):
        for gg in range(2):
            g = g2 * 2 + gg
            bb = gg
            nbb = 1 - gg
            for k in range(BLK_CH):
                c = g * BLK_CH + k
                b = k % NBUF
                nb = (k + 1) % NBUF
                if gg == 0 and k < 3:
                    @pl.when(g2 >= 1)
                    def _():
                        _wait_scatter(nb)
                else:
                    _wait_scatter(nb)
                if k == 2:
                    brow = ebrow + (g + 1) * BLK_CH
                    pltpu.sync_copy(src_hbm.at[pl.ds(brow, BLK_CH)],
                                    srcb.at[nbb])
                    pltpu.sync_copy(dst_hbm.at[pl.ds(brow, BLK_CH)],
                                    dstb.at[nbb])
                if k < BLK_CH - 1:
                    _stage(bb, k + 1, nb)
                else:
                    _stage(nbb, 0, nb)
                _process(c, bb, k, b)
        return 0

    lax.fori_loop(0, N_BLOCKS // 2, _block, 0)

    _wait_scatter((N_CHUNKS - 3) % NBUF)
    _wait_scatter((N_CHUNKS - 2) % NBUF)
    _wait_scatter((N_CHUNKS - 1) % NBUF)
    _drain_gather(N_CHUNKS % NBUF)

    plsc.subcore_barrier()

    for k in range((N_SLABS + NS - 1) // NS):
        s = sid + NS * k

        @pl.when(s < N_SLABS)
        def _():
            pltpu.async_copy(out_sh.at[pl.ds(s * CH, CH)],
                             outp_hbm.at[cid, pl.ds(s * CH, CH)], gsem0)

    for k in range((N_DEN_SLABS + NS - 1) // NS):
        s = sid + NS * k

        @pl.when(s < N_DEN_SLABS)
        def _():
            pltpu.async_copy(den_sh.at[pl.ds(s * DEN_SLAB, DEN_SLAB)],
                             denp_hbm.at[cid, pl.ds(s * DEN_SLAB, DEN_SLAB)],
                             gsem1)

    for k in range((N_SLABS + NS - 1) // NS):
        s = sid + NS * k

        @pl.when(s < N_SLABS)
        def _():
            pltpu.make_async_copy(out_sh.at[pl.ds(s * CH, CH)],
                                  outp_hbm.at[cid, pl.ds(s * CH, CH)],
                                  gsem0).wait()

    for k in range((N_DEN_SLABS + NS - 1) // NS):
        s = sid + NS * k

        @pl.when(s < N_DEN_SLABS)
        def _():
            pltpu.make_async_copy(
                den_sh.at[pl.ds(s * DEN_SLAB, DEN_SLAB)],
                denp_hbm.at[cid, pl.ds(s * DEN_SLAB, DEN_SLAB)],
                gsem1).wait()


def _sc_aggregate(src, dst, h, s1, s2):
    mesh = plsc.VectorSubcoreMesh(core_axis_name="c", subcore_axis_name="s")
    fn = pl.kernel(
        _sc_body,
        out_type=[
            jax.ShapeDtypeStruct((NC, N_PAD, D), jnp.float32),
            jax.ShapeDtypeStruct((NC, N_PAD_DEN), jnp.float32),
        ],
        mesh=mesh,
        scratch_types=[
            pltpu.VMEM((2, BLK_CH, CH), jnp.int32),
            pltpu.VMEM((2, BLK_CH, CH), jnp.int32),
            pltpu.VMEM((NBUF, CH, D), jnp.float32),
            pltpu.VMEM((NBUF, CH), jnp.float32),
            pltpu.VMEM((NBUF, CH), jnp.float32),
            pltpu.VMEM((NBUF, CH), jnp.float32),
            pltpu.VMEM((DEN_SLAB,), jnp.float32),
            pltpu.VMEM_SHARED((N_PAD, D), jnp.float32),
            pltpu.VMEM_SHARED((N_PAD_DEN,), jnp.float32),
            pltpu.SemaphoreType.DMA,
            pltpu.SemaphoreType.DMA,
            pltpu.SemaphoreType.DMA,
            pltpu.SemaphoreType.DMA,
            pltpu.SemaphoreType.DMA,
            pltpu.SemaphoreType.DMA,
            pltpu.SemaphoreType.DMA,
            pltpu.SemaphoreType.DMA,
        ],
        compiler_params=pltpu.CompilerParams(needs_layout_passes=False),
    )
    return fn(src, dst, h, s1, s2)


def _tc_combine_body(o0_ref, o1_ref, d0_ref, d1_ref, out_ref):
    den = d0_ref[...] + d1_ref[...]
    num = o0_ref[...] + o1_ref[...]
    out_ref[...] = jnp.where(den > 0.0, num / jnp.where(den > 0.0, den, 1.0),
                             0.0)


def _tc_combine(o0, o1, d0, d1):
    n_blocks = N_NODES // ROWS_BLK
    return pl.pallas_call(
        _tc_combine_body,
        grid=(n_blocks,),
        in_specs=[
            pl.BlockSpec((ROWS_BLK, D), lambda i: (i, 0)),
            pl.BlockSpec((ROWS_BLK, D), lambda i: (i, 0)),
            pl.BlockSpec((ROWS_BLK, 1), lambda i: (i, 0)),
            pl.BlockSpec((ROWS_BLK, 1), lambda i: (i, 0)),
        ],
        out_specs=pl.BlockSpec((ROWS_BLK, D), lambda i: (i, 0)),
        out_shape=jax.ShapeDtypeStruct((N_NODES, D), jnp.float32),
    )(o0, o1, d0, d1)


def kernel(node_states, edges, kernel, kernel_attention):
    w = kernel.astype(jnp.float32)
    a = kernel_attention.astype(jnp.float32)
    a_mat = jnp.pad(jnp.concatenate([a[:D], a[D:]], axis=1),
                    ((0, 0), (0, D - 2)))

    edges_i = edges.astype(jnp.int32)
    pad_pt = EDGES_PER_TILE - E_REAL_PT
    dummy = (jnp.arange(NW * pad_pt, dtype=jnp.int32) % N_NODES)
    dummy = dummy.reshape(NW, pad_pt)

    def _tile_layout(col):
        per_tile = jnp.concatenate([col.reshape(NW, E_REAL_PT), dummy], 1)
        tail = jnp.zeros(((BLK_CH + 1) * CH,), jnp.int32)
        return jnp.concatenate([per_tile.reshape(-1), tail]).reshape(-1, CH)

    src = _tile_layout(edges_i[:, 0])
    dst = _tile_layout(edges_i[:, 1])

    h, s_cat = _tc_transform(node_states.astype(jnp.float32), w, a_mat)
    s1 = s_cat[:, 0]
    s2 = s_cat[:, 1]

    outp, denp = _sc_aggregate(src, dst, h, s1, s2)

    return _tc_combine(outp[0, :N_NODES], outp[1, :N_NODES],
                       denp[0, :N_NODES, None], denp[1, :N_NODES, None])
